# baseline (speedup 1.0000x reference)
import functools

import jax
import jax.numpy as jnp
from jax import lax
from jax.experimental import pallas as pl
from jax.experimental.pallas import tpu as pltpu
from jax.experimental.pallas import tpu_sc as plsc

B = 16384
D = 128
NC = 2
NS = 16
L = 16
NW = NC * NS
BPW = B // NW
PPAD = 16
VPR = D // L
NCHUNK = 4
CH = BPW // NCHUNK
GPC = CH // L
NGRP = BPW // L


def _sc_fused(cs, params_pad, xs):
    mesh = plsc.VectorSubcoreMesh(core_axis_name="c", subcore_axis_name="s")

    @functools.partial(
        pl.kernel,
        out_type=jax.ShapeDtypeStruct((B, D), jnp.float32),
        mesh=mesh,
        scratch_types=[
            pltpu.VMEM((BPW,), jnp.int32),
            pltpu.VMEM((PPAD,), jnp.float32),
            pltpu.VMEM((BPW, D), jnp.float32),
            [pltpu.SemaphoreType.DMA] * NCHUNK,
            [pltpu.SemaphoreType.DMA] * NCHUNK,
        ],
    )
    def k(cs_hbm, p_hbm, xs_hbm, out_hbm, cs_v, p_v, x_v, sin, sout):
        wid = lax.axis_index("s") * NC + lax.axis_index("c")
        base = wid * BPW

        def in_copy(t):
            return pltpu.make_async_copy(
                xs_hbm.at[pl.ds(base + t * CH, CH)],
                x_v.at[pl.ds(t * CH, CH)],
                sin[t],
            )

        def out_copy(t):
            return pltpu.make_async_copy(
                x_v.at[pl.ds(t * CH, CH)],
                out_hbm.at[pl.ds(base + t * CH, CH)],
                sout[t],
            )

        pltpu.sync_copy(p_hbm, p_v)
        pltpu.sync_copy(cs_hbm.at[pl.ds(base, BPW)], cs_v)
        for t in range(NCHUNK):
            in_copy(t).start()
        p_vec = p_v[...]

        def grp_body(g, carry):
            for t in range(NCHUNK):
                @pl.when(g == t * GPC)
                def _():
                    in_copy(t).wait()

            r0 = g * L
            idx = cs_v[pl.ds(r0, L)]
            g16 = jnp.take_along_axis(
                p_vec, idx, axis=0, mode="promise_in_bounds"
            )
            for j in range(L):
                s = jnp.take_along_axis(
                    g16, jnp.full((L,), j, jnp.int32), axis=0,
                    mode="promise_in_bounds",
                )
                for c in range(VPR):
                    x_v[r0 + j, pl.ds(c * L, L)] = (
                        x_v[r0 + j, pl.ds(c * L, L)] * s
                    )

            for t in range(NCHUNK):
                @pl.when(g == t * GPC + (GPC - 1))
                def _():
                    out_copy(t).start()
            return carry

        lax.fori_loop(0, NGRP, grp_body, 0)

        for t in range(NCHUNK):
            out_copy(t).wait()

    return k(cs, params_pad, xs)


def kernel(cs, xs, params):
    flat = params.reshape(-1)
    p_pad = jnp.zeros((PPAD,), jnp.float32).at[: flat.shape[0]].set(flat)
    return _sc_fused(cs, p_pad, xs)


# speedup vs baseline: 1.0703x; 1.0703x over previous
import functools

import jax
import jax.numpy as jnp
from jax import lax
from jax.experimental import pallas as pl
from jax.experimental.pallas import tpu as pltpu
from jax.experimental.pallas import tpu_sc as plsc

B = 16384
D = 128
NC = 2
NS = 16
L = 16
NW = NC * NS
BPW = B // NW
PPAD = 16
VPR = D // L
NCHUNK = 4
CH = BPW // NCHUNK
GPC = CH // L
NGRP = BPW // L


def _sc_fused(cs, params_pad, xs):
    mesh = plsc.VectorSubcoreMesh(core_axis_name="c", subcore_axis_name="s")

    @functools.partial(
        pl.kernel,
        out_type=jax.ShapeDtypeStruct((B, D), jnp.float32),
        mesh=mesh,
        scratch_types=[
            pltpu.VMEM((BPW,), jnp.int32),
            pltpu.VMEM((PPAD,), jnp.float32),
            pltpu.VMEM((BPW,), jnp.float32),
            pltpu.VMEM((BPW, D), jnp.float32),
            [pltpu.SemaphoreType.DMA] * NCHUNK,
            [pltpu.SemaphoreType.DMA] * NCHUNK,
            pltpu.SemaphoreType.DMA,
            pltpu.SemaphoreType.DMA,
        ],
    )
    def k(cs_hbm, p_hbm, xs_hbm, out_hbm, cs_v, p_v, g_v, x_v, sin, sout,
          s_cs, s_p):
        wid = lax.axis_index("s") * NC + lax.axis_index("c")
        base = wid * BPW

        def in_copy(t):
            return pltpu.make_async_copy(
                xs_hbm.at[pl.ds(base + t * CH, CH)],
                x_v.at[pl.ds(t * CH, CH)],
                sin[t],
            )

        def out_copy(t):
            return pltpu.make_async_copy(
                x_v.at[pl.ds(t * CH, CH)],
                out_hbm.at[pl.ds(base + t * CH, CH)],
                sout[t],
            )

        cs_copy = pltpu.make_async_copy(
            cs_hbm.at[pl.ds(base, BPW)], cs_v, s_cs
        )
        p_copy = pltpu.make_async_copy(p_hbm, p_v, s_p)
        cs_copy.start()
        p_copy.start()
        for t in range(NCHUNK):
            in_copy(t).start()
        cs_copy.wait()
        p_copy.wait()
        p_vec = p_v[...]

        def gather_body(i, carry):
            idx = cs_v[pl.ds(i * L, L)]
            g_v[pl.ds(i * L, L)] = jnp.take_along_axis(
                p_vec, idx, axis=0, mode="promise_in_bounds"
            )
            return carry

        lax.fori_loop(0, NGRP, gather_body, 0)

        def grp_body(g, carry):
            for t in range(NCHUNK):
                @pl.when(g == t * GPC)
                def _():
                    in_copy(t).wait()

            r0 = g * L
            g16 = g_v[pl.ds(r0, L)]
            for j in range(L):
                s = jnp.take_along_axis(
                    g16, jnp.full((L,), j, jnp.int32), axis=0,
                    mode="promise_in_bounds",
                )
                for c in range(VPR):
                    x_v[r0 + j, pl.ds(c * L, L)] = (
                        x_v[r0 + j, pl.ds(c * L, L)] * s
                    )

            for t in range(NCHUNK):
                @pl.when(g == t * GPC + (GPC - 1))
                def _():
                    out_copy(t).start()
            return carry

        lax.fori_loop(0, NGRP, grp_body, 0)

        for t in range(NCHUNK):
            out_copy(t).wait()

    return k(cs, params_pad, xs)


def kernel(cs, xs, params):
    flat = params.reshape(-1)
    p_pad = jnp.zeros((PPAD,), jnp.float32).at[: flat.shape[0]].set(flat)
    return _sc_fused(cs, p_pad, xs)


# speedup vs baseline: 1.0922x; 1.0205x over previous
import functools

import jax
import jax.numpy as jnp
from jax import lax
from jax.experimental import pallas as pl
from jax.experimental.pallas import tpu as pltpu
from jax.experimental.pallas import tpu_sc as plsc

B = 16384
D = 128
NC = 2
NS = 16
L = 16
NW = NC * NS
BPW = B // NW
PPAD = 16
VPR = D // L
NCHUNK = 8
CH = BPW // NCHUNK
GPC = CH // L
NGRP = BPW // L


def _sc_fused(cs, params_pad, xs):
    mesh = plsc.VectorSubcoreMesh(core_axis_name="c", subcore_axis_name="s")

    @functools.partial(
        pl.kernel,
        out_type=jax.ShapeDtypeStruct((B, D), jnp.float32),
        mesh=mesh,
        scratch_types=[
            pltpu.VMEM((BPW,), jnp.int32),
            pltpu.VMEM((PPAD,), jnp.float32),
            pltpu.VMEM((BPW,), jnp.float32),
            pltpu.VMEM((BPW, D), jnp.float32),
            [pltpu.SemaphoreType.DMA] * NCHUNK,
            [pltpu.SemaphoreType.DMA] * NCHUNK,
            pltpu.SemaphoreType.DMA,
            pltpu.SemaphoreType.DMA,
        ],
    )
    def k(cs_hbm, p_hbm, xs_hbm, out_hbm, cs_v, p_v, g_v, x_v, sin, sout,
          s_cs, s_p):
        wid = lax.axis_index("s") * NC + lax.axis_index("c")
        base = wid * BPW

        def in_copy(t):
            return pltpu.make_async_copy(
                xs_hbm.at[pl.ds(base + t * CH, CH)],
                x_v.at[pl.ds(t * CH, CH)],
                sin[t],
            )

        def out_copy(t):
            return pltpu.make_async_copy(
                x_v.at[pl.ds(t * CH, CH)],
                out_hbm.at[pl.ds(base + t * CH, CH)],
                sout[t],
            )

        cs_copy = pltpu.make_async_copy(
            cs_hbm.at[pl.ds(base, BPW)], cs_v, s_cs
        )
        p_copy = pltpu.make_async_copy(p_hbm, p_v, s_p)
        cs_copy.start()
        p_copy.start()
        for t in range(NCHUNK):
            in_copy(t).start()
        cs_copy.wait()
        p_copy.wait()
        p_vec = p_v[...]

        def gather_body(i, carry):
            idx = cs_v[pl.ds(i * L, L)]
            g_v[pl.ds(i * L, L)] = jnp.take_along_axis(
                p_vec, idx, axis=0, mode="promise_in_bounds"
            )
            return carry

        lax.fori_loop(0, NGRP, gather_body, 0)

        def grp_body(g, carry):
            for t in range(NCHUNK):
                @pl.when(g == t * GPC)
                def _():
                    in_copy(t).wait()

            r0 = g * L
            g16 = g_v[pl.ds(r0, L)]
            for j in range(L):
                s = jnp.take_along_axis(
                    g16, jnp.full((L,), j, jnp.int32), axis=0,
                    mode="promise_in_bounds",
                )
                for c in range(VPR):
                    x_v[r0 + j, pl.ds(c * L, L)] = (
                        x_v[r0 + j, pl.ds(c * L, L)] * s
                    )

            for t in range(NCHUNK):
                @pl.when(g == t * GPC + (GPC - 1))
                def _():
                    out_copy(t).start()
            return carry

        lax.fori_loop(0, NGRP, grp_body, 0)

        for t in range(NCHUNK):
            out_copy(t).wait()

    return k(cs, params_pad, xs)


def kernel(cs, xs, params):
    flat = params.reshape(-1)
    p_pad = jnp.zeros((PPAD,), jnp.float32).at[: flat.shape[0]].set(flat)
    return _sc_fused(cs, p_pad, xs)


# speedup vs baseline: 1.1156x; 1.0215x over previous
import functools

import jax
import jax.numpy as jnp
from jax import lax
from jax.experimental import pallas as pl
from jax.experimental.pallas import tpu as pltpu
from jax.experimental.pallas import tpu_sc as plsc

B = 16384
D = 128
NC = 2
NS = 16
L = 16
NW = NC * NS
BPW = B // NW
PPAD = 16
VPR = D // L
NCHUNK = 8
CH = BPW // NCHUNK
GPC = CH // L
NGRP = BPW // L


def _sc_fused(cs, params_flat, xs):
    NP = params_flat.shape[0]
    mesh = plsc.VectorSubcoreMesh(core_axis_name="c", subcore_axis_name="s")

    @functools.partial(
        pl.kernel,
        out_type=jax.ShapeDtypeStruct((B, D), jnp.float32),
        mesh=mesh,
        scratch_types=[
            pltpu.VMEM((BPW,), jnp.int32),
            pltpu.VMEM((PPAD,), jnp.float32),
            pltpu.VMEM((BPW,), jnp.float32),
            pltpu.VMEM((BPW, D), jnp.float32),
            [pltpu.SemaphoreType.DMA] * NCHUNK,
            [pltpu.SemaphoreType.DMA] * NCHUNK,
            pltpu.SemaphoreType.DMA,
            pltpu.SemaphoreType.DMA,
        ],
    )
    def k(cs_hbm, p_hbm, xs_hbm, out_hbm, cs_v, p_v, g_v, x_v, sin, sout,
          s_cs, s_p):
        wid = lax.axis_index("s") * NC + lax.axis_index("c")
        base = wid * BPW

        def in_copy(t):
            return pltpu.make_async_copy(
                xs_hbm.at[pl.ds(base + t * CH, CH)],
                x_v.at[pl.ds(t * CH, CH)],
                sin[t],
            )

        def out_copy(t):
            return pltpu.make_async_copy(
                x_v.at[pl.ds(t * CH, CH)],
                out_hbm.at[pl.ds(base + t * CH, CH)],
                sout[t],
            )

        cs_copy = pltpu.make_async_copy(
            cs_hbm.at[pl.ds(base, BPW)], cs_v, s_cs
        )
        p_copy = pltpu.make_async_copy(p_hbm, p_v.at[pl.ds(0, NP)], s_p)
        cs_copy.start()
        p_copy.start()
        for t in range(NCHUNK):
            in_copy(t).start()
        cs_copy.wait()
        p_copy.wait()
        p_vec = p_v[...]

        def gather_body(i, carry):
            idx = cs_v[pl.ds(i * L, L)]
            g_v[pl.ds(i * L, L)] = jnp.take_along_axis(
                p_vec, idx, axis=0, mode="promise_in_bounds"
            )
            return carry

        lax.fori_loop(0, NGRP, gather_body, 0)

        def grp_body(g, carry):
            for t in range(NCHUNK):
                @pl.when(g == t * GPC)
                def _():
                    in_copy(t).wait()

            r0 = g * L
            g16 = g_v[pl.ds(r0, L)]
            for j in range(L):
                s = jnp.take_along_axis(
                    g16, jnp.full((L,), j, jnp.int32), axis=0,
                    mode="promise_in_bounds",
                )
                for c in range(VPR):
                    x_v[r0 + j, pl.ds(c * L, L)] = (
                        x_v[r0 + j, pl.ds(c * L, L)] * s
                    )

            for t in range(NCHUNK):
                @pl.when(g == t * GPC + (GPC - 1))
                def _():
                    out_copy(t).start()
            return carry

        lax.fori_loop(0, NGRP, grp_body, 0)

        for t in range(NCHUNK):
            out_copy(t).wait()

    return k(cs, params_flat, xs)


def kernel(cs, xs, params):
    return _sc_fused(cs, params.reshape(-1), xs)


# speedup vs baseline: 1.1191x; 1.0032x over previous
import functools

import jax
import jax.numpy as jnp
from jax import lax
from jax.experimental import pallas as pl
from jax.experimental.pallas import tpu as pltpu
from jax.experimental.pallas import tpu_sc as plsc

B = 16384
D = 128
NC = 2
NS = 16
L = 16
NW = NC * NS
BPW = B // NW
PPAD = 16
VPR = D // L
NCHUNK = 8
CH = BPW // NCHUNK
GPC = CH // L
NGRP = BPW // L


def _sc_fused(cs, params_flat, xs):
    NP = params_flat.shape[0]
    mesh = plsc.VectorSubcoreMesh(core_axis_name="c", subcore_axis_name="s")

    @functools.partial(
        pl.kernel,
        out_type=jax.ShapeDtypeStruct((B, D), jnp.float32),
        mesh=mesh,
        scratch_types=[
            pltpu.VMEM((BPW,), jnp.int32),
            pltpu.VMEM((PPAD,), jnp.float32),
            pltpu.VMEM((BPW,), jnp.float32),
            pltpu.VMEM((BPW, D), jnp.float32),
            [pltpu.SemaphoreType.DMA] * NCHUNK,
            pltpu.SemaphoreType.DMA,
            pltpu.SemaphoreType.DMA,
            pltpu.SemaphoreType.DMA,
        ],
    )
    def k(cs_hbm, p_hbm, xs_hbm, out_hbm, cs_v, p_v, g_v, x_v, sin, s_out,
          s_cs, s_p):
        wid = lax.axis_index("s") * NC + lax.axis_index("c")
        base = wid * BPW

        def in_copy(t):
            return pltpu.make_async_copy(
                xs_hbm.at[pl.ds(base + t * CH, CH)],
                x_v.at[pl.ds(t * CH, CH)],
                sin[t],
            )

        def out_grp_copy(r0):
            return pltpu.make_async_copy(
                x_v.at[pl.ds(r0, L)],
                out_hbm.at[pl.ds(base + r0, L)],
                s_out,
            )

        cs_copy = pltpu.make_async_copy(
            cs_hbm.at[pl.ds(base, BPW)], cs_v, s_cs
        )
        p_copy = pltpu.make_async_copy(p_hbm, p_v.at[pl.ds(0, NP)], s_p)
        cs_copy.start()
        p_copy.start()
        for t in range(NCHUNK):
            in_copy(t).start()
        cs_copy.wait()
        p_copy.wait()
        p_vec = p_v[...]

        def gather_body(i, carry):
            idx = cs_v[pl.ds(i * L, L)]
            g_v[pl.ds(i * L, L)] = jnp.take_along_axis(
                p_vec, idx, axis=0, mode="promise_in_bounds"
            )
            return carry

        lax.fori_loop(0, NGRP, gather_body, 0)

        def grp_body(g, carry):
            for t in range(NCHUNK):
                @pl.when(g == t * GPC)
                def _():
                    in_copy(t).wait()

            r0 = g * L
            g16 = g_v[pl.ds(r0, L)]
            for j in range(L):
                s = jnp.take_along_axis(
                    g16, jnp.full((L,), j, jnp.int32), axis=0,
                    mode="promise_in_bounds",
                )
                for c in range(VPR):
                    x_v[r0 + j, pl.ds(c * L, L)] = (
                        x_v[r0 + j, pl.ds(c * L, L)] * s
                    )

            out_grp_copy(r0).start()
            return carry

        lax.fori_loop(0, NGRP, grp_body, 0)

        pltpu.make_async_copy(
            x_v, out_hbm.at[pl.ds(base, BPW)], s_out
        ).wait()

    return k(cs, params_flat, xs)


def kernel(cs, xs, params):
    return _sc_fused(cs, params.reshape(-1), xs)


# speedup vs baseline: 1.1366x; 1.0156x over previous
import functools

import jax
import jax.numpy as jnp
from jax import lax
from jax.experimental import pallas as pl
from jax.experimental.pallas import tpu as pltpu
from jax.experimental.pallas import tpu_sc as plsc

B = 16384
D = 128
NC = 2
NS = 16
L = 16
NW = NC * NS
BPW = B // NW
PPAD = 16
VPR = D // L
NCHUNK = 16
CH = BPW // NCHUNK
GPC = CH // L
NGRP = BPW // L


def _sc_fused(cs, params_flat, xs):
    NP = params_flat.shape[0]
    mesh = plsc.VectorSubcoreMesh(core_axis_name="c", subcore_axis_name="s")

    @functools.partial(
        pl.kernel,
        out_type=jax.ShapeDtypeStruct((B, D), jnp.float32),
        mesh=mesh,
        scratch_types=[
            pltpu.VMEM((BPW,), jnp.int32),
            pltpu.VMEM((PPAD,), jnp.float32),
            pltpu.VMEM((BPW,), jnp.float32),
            pltpu.VMEM((BPW, D), jnp.float32),
            [pltpu.SemaphoreType.DMA] * NCHUNK,
            pltpu.SemaphoreType.DMA,
            pltpu.SemaphoreType.DMA,
            pltpu.SemaphoreType.DMA,
        ],
    )
    def k(cs_hbm, p_hbm, xs_hbm, out_hbm, cs_v, p_v, g_v, x_v, sin, s_out,
          s_cs, s_p):
        wid = lax.axis_index("s") * NC + lax.axis_index("c")
        base = wid * BPW

        def in_copy(t):
            return pltpu.make_async_copy(
                xs_hbm.at[pl.ds(base + t * CH, CH)],
                x_v.at[pl.ds(t * CH, CH)],
                sin[t],
            )

        def out_grp_copy(r0):
            return pltpu.make_async_copy(
                x_v.at[pl.ds(r0, L)],
                out_hbm.at[pl.ds(base + r0, L)],
                s_out,
            )

        cs_copy = pltpu.make_async_copy(
            cs_hbm.at[pl.ds(base, BPW)], cs_v, s_cs
        )
        p_copy = pltpu.make_async_copy(p_hbm, p_v.at[pl.ds(0, NP)], s_p)
        cs_copy.start()
        p_copy.start()
        for t in range(NCHUNK):
            in_copy(t).start()
        cs_copy.wait()
        p_copy.wait()
        p_vec = p_v[...]

        def gather_body(i, carry):
            idx = cs_v[pl.ds(i * L, L)]
            g_v[pl.ds(i * L, L)] = jnp.take_along_axis(
                p_vec, idx, axis=0, mode="promise_in_bounds"
            )
            return carry

        lax.fori_loop(0, NGRP, gather_body, 0)

        def grp_body(g, carry):
            for t in range(NCHUNK):
                @pl.when(g == t * GPC)
                def _():
                    in_copy(t).wait()

            r0 = g * L
            g16 = g_v[pl.ds(r0, L)]
            for j in range(L):
                s = jnp.take_along_axis(
                    g16, jnp.full((L,), j, jnp.int32), axis=0,
                    mode="promise_in_bounds",
                )
                for c in range(VPR):
                    x_v[r0 + j, pl.ds(c * L, L)] = (
                        x_v[r0 + j, pl.ds(c * L, L)] * s
                    )

            out_grp_copy(r0).start()
            return carry

        lax.fori_loop(0, NGRP, grp_body, 0)

        pltpu.make_async_copy(
            x_v, out_hbm.at[pl.ds(base, BPW)], s_out
        ).wait()

    return k(cs, params_flat, xs)


def kernel(cs, xs, params):
    return _sc_fused(cs, params.reshape(-1), xs)
